# baseline (speedup 1.0000x reference)
import jax
import jax.numpy as jnp
from jax.experimental import pallas as pl
from jax.experimental.pallas import tpu as pltpu

N_EXP = 8
D_MODEL = 768
BT = 2048


def _gate_body(tok_ref, emb_ref, w_ref, idx_ref):
    emb = emb_ref[...]
    norm = jnp.sqrt(jnp.sum(emb * emb, axis=-1, keepdims=True))
    wn = (emb / jnp.maximum(norm, 1e-12)).astype(jnp.bfloat16)
    tok = tok_ref[...]
    tnorm = jnp.sqrt(jnp.sum(tok * tok, axis=-1, keepdims=True))
    nt = (tok / jnp.maximum(tnorm, 1e-12)).astype(jnp.bfloat16)
    simsT = jax.lax.dot_general(
        wn, nt, dimension_numbers=(((1,), (1,)), ((), ())),
        preferred_element_type=jnp.float32)
    m = jnp.max(simsT, axis=0, keepdims=True)
    eiota = jax.lax.broadcasted_iota(jnp.int32, simsT.shape, 0)
    idxT = jnp.min(jnp.where(simsT == m, eiota, N_EXP), axis=0, keepdims=True)
    wT = (eiota == idxT).astype(jnp.float32)
    w_ref[...] = wT.T
    idx_ref[...] = idxT.T


@jax.jit
def kernel(language_token, routing_embeddings):
    n_tokens = language_token.shape[0]
    steps = n_tokens // BT
    weights, indices = pl.pallas_call(
        _gate_body,
        grid=(steps,),
        in_specs=[
            pl.BlockSpec((BT, D_MODEL), lambda i: (i, 0)),
            pl.BlockSpec((N_EXP, D_MODEL), lambda i: (0, 0)),
        ],
        out_specs=[
            pl.BlockSpec((BT, N_EXP), lambda i: (i, 0)),
            pl.BlockSpec((BT, 1), lambda i: (i, 0)),
        ],
        out_shape=[
            jax.ShapeDtypeStruct((n_tokens, N_EXP), jnp.float32),
            jax.ShapeDtypeStruct((n_tokens, 1), jnp.int32),
        ],
    )(language_token, routing_embeddings)
    return (weights, indices)
